# baseline (speedup 1.0000x reference)
import functools

import jax
import jax.numpy as jnp
from jax import lax
from jax.experimental import pallas as pl
from jax.experimental.pallas import tpu as pltpu
from jax.experimental.pallas import tpu_sc as plsc

_N = 10000
_NC = 2
_NS = 16
_NW = _NC * _NS
_K = 80
_DEG_PAD = 10240
_NPAD = 10240
_RSTG = 128
_F = 64
_NBUF = 5
_K2 = 128

_mesh = functools.partial(
    plsc.VectorSubcoreMesh,
    core_axis_name="c", subcore_axis_name="s",
    num_cores=_NC, num_subcores=_NS)


@functools.lru_cache(maxsize=None)
def _deg_kernel(ept):
    @functools.partial(
        pl.kernel,
        out_type=jax.ShapeDtypeStruct((_NW, _DEG_PAD // 16, 16), jnp.float32),
        mesh=_mesh(),
        scratch_types=[
            pltpu.VMEM((ept,), jnp.int32),
            pltpu.VMEM((_DEG_PAD // 16, 16), jnp.float32),
        ],
        compiler_params=pltpu.CompilerParams(needs_layout_passes=False),
    )
    def deg_k(dst_hbm, out_hbm, dstv, degv):
        cid = lax.axis_index("c")
        sid = lax.axis_index("s")
        wid = sid * _NC + cid
        pltpu.sync_copy(dst_hbm.at[wid], dstv)
        zero16 = jnp.zeros((16,), jnp.float32)

        def zbody(i, carry):
            degv[i, :] = zero16
            return carry

        lax.fori_loop(0, _DEG_PAD // 16, zbody, 0)
        one16 = jnp.ones((16,), jnp.float32)

        def body(i, carry):
            idx = dstv[pl.ds(pl.multiple_of(i * 16, 16), 16)]
            row = lax.shift_right_logical(idx, 4)
            col = lax.bitwise_and(idx, 15)
            plsc.addupdate_scatter(degv, [row, col], one16)
            return carry

        lax.fori_loop(0, ept // 16, body, 0)
        pltpu.sync_copy(degv, out_hbm.at[wid])

    return deg_k


@functools.lru_cache(maxsize=None)
def _prop_kernel(nchunk):
    rpt = _NPAD // _NS

    @functools.partial(
        pl.kernel,
        out_type=jax.ShapeDtypeStruct((_NC, _NPAD, _F), jnp.float32),
        mesh=_mesh(),
        scratch_types=[
            pltpu.VMEM((nchunk, _K), jnp.int32),
            pltpu.VMEM((nchunk, _K), jnp.int32),
            pltpu.VMEM((_NBUF, _K, _F), jnp.float32),
            pltpu.VMEM((_RSTG, _F), jnp.float32),
            pltpu.VMEM_SHARED((_NPAD, _F), jnp.float32),
            [pltpu.SemaphoreType.DMA] * _NBUF,
            [pltpu.SemaphoreType.DMA] * _NBUF,
        ],
        compiler_params=pltpu.CompilerParams(
            needs_layout_passes=False, use_tc_tiling_on_sc=False),
    )
    def prop_k(g_hbm, src_hbm, dst_hbm, out_hbm, srcv, dstv, buf, stg, acc,
               gsem, ssem):
        cid = lax.axis_index("c")
        sid = lax.axis_index("s")
        wid = sid * _NC + cid
        pltpu.sync_copy(src_hbm.at[wid], srcv)
        pltpu.sync_copy(dst_hbm.at[wid], dstv)

        zero16 = jnp.zeros((16,), jnp.float32)

        def zbody(i, carry):
            for j in range(_F // 16):
                stg[i, pl.ds(j * 16, 16)] = zero16
            return carry

        lax.fori_loop(0, _RSTG, zbody, 0)
        base = sid * rpt
        for q in range(rpt // _RSTG):
            pltpu.sync_copy(stg, acc.at[pl.ds(base + q * _RSTG, _RSTG)])
        plsc.subcore_barrier()

        for b in range(_NBUF):
            pltpu.async_copy(g_hbm.at[srcv.at[b]], buf.at[b], gsem[b])

        def outer(gidx, carry):
            jbase = gidx * _NBUF
            for b in range(_NBUF):
                j = jbase + b
                pltpu.make_async_copy(
                    g_hbm.at[srcv.at[j]], buf.at[b], gsem[b]).wait()
                pltpu.async_copy(
                    buf.at[b], acc.at[dstv.at[j]], ssem[b], add=True)
            for b in range(_NBUF):
                j = jbase + b
                pltpu.make_async_copy(
                    buf.at[b], acc.at[dstv.at[j]], ssem[b]).wait()
                nxt = j + _NBUF

                @pl.when(nxt < nchunk)
                def _():
                    pltpu.async_copy(
                        g_hbm.at[srcv.at[nxt]], buf.at[b], gsem[b])
            return carry

        lax.fori_loop(0, nchunk // _NBUF, outer, 0)
        plsc.subcore_barrier()

        for q in range(rpt // _RSTG):
            r0 = base + q * _RSTG
            pltpu.sync_copy(acc.at[pl.ds(r0, _RSTG)], stg)
            pltpu.sync_copy(stg, out_hbm.at[cid, pl.ds(r0, _RSTG)])

    return prop_k


@functools.lru_cache(maxsize=None)
def _prop2_kernel(nchunk):
    rpt = _NPAD // _NS

    @functools.partial(
        pl.kernel,
        out_type=jax.ShapeDtypeStruct((_NC, _NPAD, _F), jnp.float32),
        mesh=_mesh(),
        scratch_types=[
            pltpu.VMEM((nchunk, _K2), jnp.int32),
            pltpu.VMEM((nchunk, _K2), jnp.int32),
            pltpu.VMEM((_NBUF, _K2, _F), jnp.float32),
            pltpu.VMEM((_RSTG, _F), jnp.float32),
            pltpu.VMEM_SHARED((_NPAD, _F), jnp.float32),
            [pltpu.SemaphoreType.DMA] * _NBUF,
            [pltpu.SemaphoreType.DMA] * _NBUF,
        ],
        compiler_params=pltpu.CompilerParams(
            needs_layout_passes=False, use_tc_tiling_on_sc=False),
    )
    def prop_k(g_hbm, src_hbm, dst_hbm, out_hbm, srcv, dstv, buf, stg, acc,
               gsem, ssem):
        cid = lax.axis_index("c")
        sid = lax.axis_index("s")
        pltpu.sync_copy(src_hbm.at[sid], srcv)
        pltpu.sync_copy(dst_hbm.at[sid], dstv)
        gh = g_hbm.at[cid]

        zero16 = jnp.zeros((16,), jnp.float32)

        def zbody(i, carry):
            for j in range(_F // 16):
                stg[i, pl.ds(j * 16, 16)] = zero16
            return carry

        lax.fori_loop(0, _RSTG, zbody, 0)
        base = sid * rpt
        for q in range(rpt // _RSTG):
            pltpu.sync_copy(stg, acc.at[pl.ds(base + q * _RSTG, _RSTG)])
        plsc.subcore_barrier()

        for b in range(_NBUF):
            pltpu.async_copy(gh.at[srcv.at[b]], buf.at[b], gsem[b])

        def outer(gidx, carry):
            jbase = gidx * _NBUF
            for b in range(_NBUF):
                j = jbase + b
                pltpu.make_async_copy(
                    gh.at[srcv.at[j]], buf.at[b], gsem[b]).wait()
                pltpu.async_copy(
                    buf.at[b], acc.at[dstv.at[j]], ssem[b], add=True)
            for b in range(_NBUF):
                j = jbase + b
                pltpu.make_async_copy(
                    buf.at[b], acc.at[dstv.at[j]], ssem[b]).wait()
                nxt = j + _NBUF

                @pl.when(nxt < nchunk)
                def _():
                    pltpu.async_copy(gh.at[srcv.at[nxt]], buf.at[b], gsem[b])
            return carry

        lax.fori_loop(0, nchunk // _NBUF, outer, 0)
        plsc.subcore_barrier()

        for q in range(rpt // _RSTG):
            r0 = base + q * _RSTG
            pltpu.sync_copy(acc.at[pl.ds(r0, _RSTG)], stg)
            pltpu.sync_copy(stg, out_hbm.at[cid, pl.ds(r0, _RSTG)])

    return prop_k


def _silu(t):
    return t * (1.0 / (1.0 + jnp.exp(-t)))


def _tc_prologue(degt, x, w1):
    def body(degt_ref, x_ref, w1_ref, g_ref, dinv_ref):
        deg = jnp.sum(degt_ref[...], axis=1, keepdims=True)[:_N] + 1.0
        dinv = lax.rsqrt(deg)
        dinv_ref[...] = dinv
        r = dinv * jnp.dot(
            x_ref[...], w1_ref[...], preferred_element_type=jnp.float32)
        g_ref[0] = r[:, :_F]
        g_ref[1] = r[:, _F:]

    return pl.pallas_call(
        body,
        out_shape=(
            jax.ShapeDtypeStruct((2, _N, _F), jnp.float32),
            jax.ShapeDtypeStruct((_N, 1), jnp.float32),
        ),
    )(degt, x, w1)


def _tc_layer(s, g, dinv, b, w, split_out):
    def body(s_ref, g_ref, dinv_ref, b_ref, w_ref, out_ref):
        dv = dinv_ref[...]
        ta = dv * (s_ref[0, :_N] + g_ref[0]) + b_ref[:, :_F]
        tb = dv * (s_ref[1, :_N] + g_ref[1]) + b_ref[:, _F:]
        r = dv * (
            jnp.dot(_silu(ta), w_ref[:_F], preferred_element_type=jnp.float32)
            + jnp.dot(_silu(tb), w_ref[_F:], preferred_element_type=jnp.float32))
        if split_out:
            out_ref[0] = r[:, :_F]
            out_ref[1] = r[:, _F:]
        else:
            out_ref[...] = r

    out_shape = ((2, _N, _F) if split_out else (_N, w.shape[1]))
    return pl.pallas_call(
        body,
        out_shape=jax.ShapeDtypeStruct(out_shape, jnp.float32),
    )(s, g, dinv, b, w)


def _tc_final(s, g, dinv, b):
    def body(s_ref, g_ref, dinv_ref, b_ref, out_ref):
        t = dinv_ref[...] * (s_ref[0, :_N] + s_ref[1, :_N] + g_ref[...]) + b_ref[...]
        m = jnp.max(t, axis=1, keepdims=True)
        sh = t - m
        lse = jnp.log(jnp.sum(jnp.exp(sh), axis=1, keepdims=True))
        out_ref[...] = sh - lse

    return pl.pallas_call(
        body,
        out_shape=jax.ShapeDtypeStruct(g.shape, jnp.float32),
    )(s, g, dinv, b)


def kernel(x, edge_index, W1, b1, W2, b2, W3, b3):
    E = edge_index.shape[1]
    assert E % (_NW * _K) == 0
    ept = E // _NW
    nchunk = ept // _K
    src = edge_index[0].reshape(_NW, nchunk, _K)
    dst = edge_index[1].reshape(_NW, nchunk, _K)
    dst_flat = edge_index[1].reshape(_NW, ept)

    epad = -(-E // (_NS * _K2 * _NBUF)) * (_NS * _K2 * _NBUF)
    npadd = epad - E
    psrc = jnp.concatenate([edge_index[0], jnp.zeros((npadd,), jnp.int32)])
    pdst = jnp.concatenate(
        [edge_index[1],
         _N + (jnp.arange(npadd, dtype=jnp.int32) % (_NPAD - _N))])
    nchunk2 = epad // (_NS * _K2)
    src2 = psrc.reshape(_NS, nchunk2, _K2)
    dst2 = pdst.reshape(_NS, nchunk2, _K2)

    degp = _deg_kernel(ept)(dst_flat)
    degt = degp.reshape(_NW, _DEG_PAD).T
    g1, dinv = _tc_prologue(degt, x, W1)
    prop2 = _prop2_kernel(nchunk2)
    s1 = prop2(g1, src2, dst2)
    g2 = _tc_layer(s1, g1, dinv, b1.reshape(1, -1), W2, split_out=True)
    s2 = prop2(g2, src2, dst2)
    g3 = _tc_layer(s2, g2, dinv, b2.reshape(1, -1), W3, split_out=False)
    s3 = _prop_kernel(nchunk)(g3, src, dst)
    return _tc_final(s3, g3, dinv, b3.reshape(1, -1))


# speedup vs baseline: 1.8025x; 1.8025x over previous
import functools

import jax
import jax.numpy as jnp
from jax import lax
from jax.experimental import pallas as pl
from jax.experimental.pallas import tpu as pltpu
from jax.experimental.pallas import tpu_sc as plsc

_N = 10000
_NC = 2
_NS = 16
_NW = _NC * _NS
_K = 80
_DEG_PAD = 10240
_NPAD = 10240
_RSTG = 128
_F = 64
_NBUF = 5
_K2 = 80

_mesh = functools.partial(
    plsc.VectorSubcoreMesh,
    core_axis_name="c", subcore_axis_name="s",
    num_cores=_NC, num_subcores=_NS)


@functools.lru_cache(maxsize=None)
def _deg_kernel(ept):
    @functools.partial(
        pl.kernel,
        out_type=jax.ShapeDtypeStruct((_NW, _DEG_PAD // 16, 16), jnp.float32),
        mesh=_mesh(),
        scratch_types=[
            pltpu.VMEM((ept,), jnp.int32),
            pltpu.VMEM((_DEG_PAD // 16, 16), jnp.float32),
        ],
        compiler_params=pltpu.CompilerParams(needs_layout_passes=False),
    )
    def deg_k(dst_hbm, out_hbm, dstv, degv):
        cid = lax.axis_index("c")
        sid = lax.axis_index("s")
        wid = sid * _NC + cid
        pltpu.sync_copy(dst_hbm.at[wid], dstv)
        zero16 = jnp.zeros((16,), jnp.float32)

        def zbody(i, carry):
            degv[i, :] = zero16
            return carry

        lax.fori_loop(0, _DEG_PAD // 16, zbody, 0)
        one16 = jnp.ones((16,), jnp.float32)

        def body(i, carry):
            idx = dstv[pl.ds(pl.multiple_of(i * 16, 16), 16)]
            row = lax.shift_right_logical(idx, 4)
            col = lax.bitwise_and(idx, 15)
            plsc.addupdate_scatter(degv, [row, col], one16)
            return carry

        lax.fori_loop(0, ept // 16, body, 0)
        pltpu.sync_copy(degv, out_hbm.at[wid])

    return deg_k


@functools.lru_cache(maxsize=None)
def _prop_kernel(nchunk):
    rpt = _NPAD // _NS

    @functools.partial(
        pl.kernel,
        out_type=jax.ShapeDtypeStruct((_NC, _NPAD, _F), jnp.float32),
        mesh=_mesh(),
        scratch_types=[
            pltpu.VMEM((nchunk, _K), jnp.int32),
            pltpu.VMEM((nchunk, _K), jnp.int32),
            pltpu.VMEM((_NBUF, _K, _F), jnp.float32),
            pltpu.VMEM((_RSTG, _F), jnp.float32),
            pltpu.VMEM_SHARED((_NPAD, _F), jnp.float32),
            [pltpu.SemaphoreType.DMA] * _NBUF,
            [pltpu.SemaphoreType.DMA] * _NBUF,
        ],
        compiler_params=pltpu.CompilerParams(
            needs_layout_passes=False, use_tc_tiling_on_sc=False),
    )
    def prop_k(g_hbm, src_hbm, dst_hbm, out_hbm, srcv, dstv, buf, stg, acc,
               gsem, ssem):
        cid = lax.axis_index("c")
        sid = lax.axis_index("s")
        wid = sid * _NC + cid
        pltpu.sync_copy(src_hbm.at[wid], srcv)
        pltpu.sync_copy(dst_hbm.at[wid], dstv)

        zero16 = jnp.zeros((16,), jnp.float32)

        def zbody(i, carry):
            for j in range(_F // 16):
                stg[i, pl.ds(j * 16, 16)] = zero16
            return carry

        lax.fori_loop(0, _RSTG, zbody, 0)
        base = sid * rpt
        for q in range(rpt // _RSTG):
            pltpu.sync_copy(stg, acc.at[pl.ds(base + q * _RSTG, _RSTG)])
        plsc.subcore_barrier()

        for b in range(_NBUF):
            pltpu.async_copy(g_hbm.at[srcv.at[b]], buf.at[b], gsem[b])

        def outer(gidx, carry):
            jbase = gidx * _NBUF
            for b in range(_NBUF):
                j = jbase + b
                pltpu.make_async_copy(
                    g_hbm.at[srcv.at[j]], buf.at[b], gsem[b]).wait()
                pltpu.async_copy(
                    buf.at[b], acc.at[dstv.at[j]], ssem[b], add=True)
            for b in range(_NBUF):
                j = jbase + b
                pltpu.make_async_copy(
                    buf.at[b], acc.at[dstv.at[j]], ssem[b]).wait()
                nxt = j + _NBUF

                @pl.when(nxt < nchunk)
                def _():
                    pltpu.async_copy(
                        g_hbm.at[srcv.at[nxt]], buf.at[b], gsem[b])
            return carry

        lax.fori_loop(0, nchunk // _NBUF, outer, 0)
        plsc.subcore_barrier()

        for q in range(rpt // _RSTG):
            r0 = base + q * _RSTG
            pltpu.sync_copy(acc.at[pl.ds(r0, _RSTG)], stg)
            pltpu.sync_copy(stg, out_hbm.at[cid, pl.ds(r0, _RSTG)])

    return prop_k


@functools.lru_cache(maxsize=None)
def _prop2_kernel(nchunk):
    rpt = _NPAD // _NS

    @functools.partial(
        pl.kernel,
        out_type=jax.ShapeDtypeStruct((_NC, _NPAD, _F), jnp.float32),
        mesh=_mesh(),
        scratch_types=[
            pltpu.VMEM((nchunk, _K2), jnp.int32),
            pltpu.VMEM((nchunk, _K2), jnp.int32),
            pltpu.VMEM((_NBUF, _K2, _F), jnp.float32),
            pltpu.VMEM((_RSTG, _F), jnp.float32),
            pltpu.VMEM_SHARED((_NPAD, _F), jnp.float32),
            [pltpu.SemaphoreType.DMA] * _NBUF,
            [pltpu.SemaphoreType.DMA] * _NBUF,
        ],
        compiler_params=pltpu.CompilerParams(
            needs_layout_passes=False, use_tc_tiling_on_sc=False),
    )
    def prop_k(g_hbm, src_hbm, dst_hbm, out_hbm, srcv, dstv, buf, stg, acc,
               gsem, ssem):
        cid = lax.axis_index("c")
        sid = lax.axis_index("s")
        pltpu.sync_copy(src_hbm.at[sid], srcv)
        pltpu.sync_copy(dst_hbm.at[sid], dstv)
        gh = g_hbm.at[cid]

        zero16 = jnp.zeros((16,), jnp.float32)

        def zbody(i, carry):
            for j in range(_F // 16):
                stg[i, pl.ds(j * 16, 16)] = zero16
            return carry

        lax.fori_loop(0, _RSTG, zbody, 0)
        base = sid * rpt
        for q in range(rpt // _RSTG):
            pltpu.sync_copy(stg, acc.at[pl.ds(base + q * _RSTG, _RSTG)])
        plsc.subcore_barrier()

        for b in range(_NBUF):
            pltpu.async_copy(gh.at[srcv.at[b]], buf.at[b], gsem[b])

        def outer(gidx, carry):
            jbase = gidx * _NBUF
            for b in range(_NBUF):
                j = jbase + b
                pltpu.make_async_copy(
                    gh.at[srcv.at[j]], buf.at[b], gsem[b]).wait()
                pltpu.async_copy(
                    buf.at[b], acc.at[dstv.at[j]], ssem[b], add=True)
            for b in range(_NBUF):
                j = jbase + b
                pltpu.make_async_copy(
                    buf.at[b], acc.at[dstv.at[j]], ssem[b]).wait()
                nxt = j + _NBUF

                @pl.when(nxt < nchunk)
                def _():
                    pltpu.async_copy(gh.at[srcv.at[nxt]], buf.at[b], gsem[b])
            return carry

        lax.fori_loop(0, nchunk // _NBUF, outer, 0)
        plsc.subcore_barrier()

        for q in range(rpt // _RSTG):
            r0 = base + q * _RSTG
            pltpu.sync_copy(acc.at[pl.ds(r0, _RSTG)], stg)
            pltpu.sync_copy(stg, out_hbm.at[cid, pl.ds(r0, _RSTG)])

    return prop_k


def _silu(t):
    return t * (1.0 / (1.0 + jnp.exp(-t)))


def _tc_prologue(degt, x, w1):
    def body(degt_ref, x_ref, w1_ref, g_ref, dinv_ref):
        deg = jnp.sum(degt_ref[...], axis=1, keepdims=True)[:_N] + 1.0
        dinv = lax.rsqrt(deg)
        dinv_ref[...] = dinv
        r = dinv * jnp.dot(
            x_ref[...], w1_ref[...], preferred_element_type=jnp.float32)
        g_ref[0] = r[:, :_F]
        g_ref[1] = r[:, _F:]

    return pl.pallas_call(
        body,
        out_shape=(
            jax.ShapeDtypeStruct((2, _N, _F), jnp.float32),
            jax.ShapeDtypeStruct((_N, 1), jnp.float32),
        ),
    )(degt, x, w1)


def _tc_layer(s, g, dinv, b, w, split_out):
    def body(s_ref, g_ref, dinv_ref, b_ref, w_ref, out_ref):
        dv = dinv_ref[...]
        ta = dv * (s_ref[0, :_N] + g_ref[0]) + b_ref[:, :_F]
        tb = dv * (s_ref[1, :_N] + g_ref[1]) + b_ref[:, _F:]
        r = dv * (
            jnp.dot(_silu(ta), w_ref[:_F], preferred_element_type=jnp.float32)
            + jnp.dot(_silu(tb), w_ref[_F:], preferred_element_type=jnp.float32))
        if split_out:
            out_ref[0] = r[:, :_F]
            out_ref[1] = r[:, _F:]
        else:
            out_ref[...] = r

    out_shape = ((2, _N, _F) if split_out else (_N, w.shape[1]))
    return pl.pallas_call(
        body,
        out_shape=jax.ShapeDtypeStruct(out_shape, jnp.float32),
    )(s, g, dinv, b, w)


def _tc_final(s, g, dinv, b):
    def body(s_ref, g_ref, dinv_ref, b_ref, out_ref):
        t = dinv_ref[...] * (s_ref[0, :_N] + s_ref[1, :_N] + g_ref[...]) + b_ref[...]
        m = jnp.max(t, axis=1, keepdims=True)
        sh = t - m
        lse = jnp.log(jnp.sum(jnp.exp(sh), axis=1, keepdims=True))
        out_ref[...] = sh - lse

    return pl.pallas_call(
        body,
        out_shape=jax.ShapeDtypeStruct(g.shape, jnp.float32),
    )(s, g, dinv, b)


def kernel(x, edge_index, W1, b1, W2, b2, W3, b3):
    E = edge_index.shape[1]
    assert E % (_NW * _K) == 0
    ept = E // _NW
    nchunk = ept // _K
    src = edge_index[0].reshape(_NW, nchunk, _K)
    dst = edge_index[1].reshape(_NW, nchunk, _K)
    dst_flat = edge_index[1].reshape(_NW, ept)

    epad = -(-E // (_NS * _K2 * _NBUF)) * (_NS * _K2 * _NBUF)
    npadd = epad - E
    psrc = jnp.concatenate([edge_index[0], jnp.zeros((npadd,), jnp.int32)])
    pdst = jnp.concatenate(
        [edge_index[1],
         _N + (jnp.arange(npadd, dtype=jnp.int32) % (_NPAD - _N))])
    nchunk2 = epad // (_NS * _K2)
    src2 = psrc.reshape(_NS, nchunk2, _K2)
    dst2 = pdst.reshape(_NS, nchunk2, _K2)

    degp = _deg_kernel(ept)(dst_flat)
    degt = degp.reshape(_NW, _DEG_PAD).T
    g1, dinv = _tc_prologue(degt, x, W1)
    prop2 = _prop2_kernel(nchunk2)
    s1 = prop2(g1, src2, dst2)
    g2 = _tc_layer(s1, g1, dinv, b1.reshape(1, -1), W2, split_out=True)
    s2 = prop2(g2, src2, dst2)
    g3 = _tc_layer(s2, g2, dinv, b2.reshape(1, -1), W3, split_out=False)
    s3 = _prop_kernel(nchunk)(g3, src, dst)
    return _tc_final(s3, g3, dinv, b3.reshape(1, -1))
